# Initial kernel scaffold; baseline (speedup 1.0000x reference)
import jax
import jax.numpy as jnp
from jax.experimental import pallas as pl


def kernel(raw_features, dis_ids, table, gamma, beta):
    raise NotImplementedError("write your pallas kernel here")


# speedup vs baseline: 6.3084x; 6.3084x over previous
import functools

import jax
import jax.numpy as jnp
from jax import lax
from jax.experimental import pallas as pl
from jax.experimental.pallas import tpu as pltpu
from jax.experimental.pallas import tpu_sc as plsc

VOCAB = 100000
H = 64
B = 4096
L = 50
EPS = 1e-12

NC = 2
NS = 16
NW = NC * NS
ROWS_PER_W = B // NW
CB = 2
NCHUNK = ROWS_PER_W // CB
IDX_PER_CHUNK = CB * L


def _lane_total(v):
    lane15 = jnp.full((16,), 15, jnp.int32)
    return plsc.cumsum(v).at[lane15].get(mode="promise_in_bounds")


def _rsqrt(v):
    i = plsc.bitcast(v, jnp.int32)
    y = plsc.bitcast(jnp.int32(0x5F3759DF) - (i >> 1), jnp.float32)
    half = v * 0.5
    for _ in range(3):
        y = y * (1.5 - half * y * y)
    return y


def _body(table_hbm, idx_hbm, gamma_hbm, beta_hbm, out_hbm,
          idx_v, buf_v, out_v, gamma_v, beta_v, sem):
    wid = lax.axis_index("s") * NC + lax.axis_index("c")

    pltpu.sync_copy(idx_hbm.at[pl.ds(wid * NCHUNK, NCHUNK)], idx_v)
    pltpu.sync_copy(gamma_hbm, gamma_v)
    pltpu.sync_copy(beta_hbm, beta_v)
    gvec = [gamma_v[pl.ds(k * 16, 16)] for k in range(4)]
    bvec = [beta_v[pl.ds(k * 16, 16)] for k in range(4)]

    @pl.loop(0, NCHUNK)
    def _chunk(c):
        pltpu.async_copy(table_hbm.at[idx_v.at[c]], buf_v, sem).wait()
        for j in range(CB):
            acc = [buf_v[j * L, pl.ds(k * 16, 16)] for k in range(4)]
            for l in range(1, L):
                for k in range(4):
                    acc[k] = acc[k] + buf_v[j * L + l, pl.ds(k * 16, 16)]
            s = (acc[0] + acc[1]) + (acc[2] + acc[3])
            mean = _lane_total(s) * (1.0 / H)
            cent = [a - mean for a in acc]
            sq = ((cent[0] * cent[0] + cent[1] * cent[1])
                  + (cent[2] * cent[2] + cent[3] * cent[3]))
            var = _lane_total(sq) * (1.0 / H)
            rstd = _rsqrt(var + EPS)
            base = c * (CB * H) + j * H
            for k in range(4):
                out_v[pl.ds(base + k * 16, 16)] = cent[k] * rstd * gvec[k] + bvec[k]

    pltpu.sync_copy(out_v, out_hbm.at[pl.ds(wid * ROWS_PER_W * H, ROWS_PER_W * H)])


@functools.partial(jax.jit, static_argnames=())
def kernel(raw_features, dis_ids, table, gamma, beta):
    del dis_ids
    idx2d = raw_features.reshape(B * L // IDX_PER_CHUNK, IDX_PER_CHUNK)
    run = pl.kernel(
        _body,
        out_type=jax.ShapeDtypeStruct((B * H,), jnp.float32),
        mesh=plsc.VectorSubcoreMesh(core_axis_name="c", subcore_axis_name="s"),
        scratch_types=[
            pltpu.VMEM((NCHUNK, IDX_PER_CHUNK), jnp.int32),
            pltpu.VMEM((IDX_PER_CHUNK, H), jnp.float32),
            pltpu.VMEM((ROWS_PER_W * H,), jnp.float32),
            pltpu.VMEM((H,), jnp.float32),
            pltpu.VMEM((H,), jnp.float32),
            pltpu.SemaphoreType.DMA,
        ],
        compiler_params=pltpu.CompilerParams(
            needs_layout_passes=False, use_tc_tiling_on_sc=False),
    )
    out = run(table, idx2d, gamma, beta)
    return out.reshape(B, H)


# speedup vs baseline: 7.2610x; 1.1510x over previous
import functools

import jax
import jax.numpy as jnp
from jax import lax
from jax.experimental import pallas as pl
from jax.experimental.pallas import tpu as pltpu
from jax.experimental.pallas import tpu_sc as plsc

VOCAB = 100000
H = 64
B = 4096
L = 50
EPS = 1e-12

NC = 2
NS = 16
NW = NC * NS
ROWS_PER_W = B // NW
CB = 2
NCHUNK = ROWS_PER_W // CB
IDX_PER_CHUNK = CB * L


def _lane_total(v):
    lane15 = jnp.full((16,), 15, jnp.int32)
    return plsc.cumsum(v).at[lane15].get(mode="promise_in_bounds")


def _rsqrt(v):
    i = plsc.bitcast(v, jnp.int32)
    y = plsc.bitcast(jnp.int32(0x5F3759DF) - (i >> 1), jnp.float32)
    half = v * 0.5
    for _ in range(3):
        y = y * (1.5 - half * y * y)
    return y


NBUF = 4


def _reduce_rows(buf, c, out_v, gvec, bvec):
    for j in range(CB):
        acc = [buf[j * L, pl.ds(k * 16, 16)] for k in range(4)]
        for l in range(1, L):
            for k in range(4):
                acc[k] = acc[k] + buf[j * L + l, pl.ds(k * 16, 16)]
        s = (acc[0] + acc[1]) + (acc[2] + acc[3])
        mean = _lane_total(s) * (1.0 / H)
        cent = [a - mean for a in acc]
        sq = ((cent[0] * cent[0] + cent[1] * cent[1])
              + (cent[2] * cent[2] + cent[3] * cent[3]))
        var = _lane_total(sq) * (1.0 / H)
        rstd = _rsqrt(var + EPS)
        base = c * (CB * H) + j * H
        for k in range(4):
            out_v[pl.ds(base + k * 16, 16)] = cent[k] * rstd * gvec[k] + bvec[k]


def _body(table_hbm, idx_hbm, gamma_hbm, beta_hbm, out_hbm,
          idx_v, b0, b1, b2, b3, out_v, gamma_v, beta_v,
          s0, s1, s2, s3):
    wid = lax.axis_index("s") * NC + lax.axis_index("c")
    bufs = (b0, b1, b2, b3)
    sems = (s0, s1, s2, s3)

    pltpu.sync_copy(idx_hbm.at[pl.ds(wid * NCHUNK, NCHUNK)], idx_v)
    pltpu.sync_copy(gamma_hbm, gamma_v)
    pltpu.sync_copy(beta_hbm, beta_v)
    gvec = [gamma_v[pl.ds(k * 16, 16)] for k in range(4)]
    bvec = [beta_v[pl.ds(k * 16, 16)] for k in range(4)]

    for b in range(NBUF):
        pltpu.async_copy(table_hbm.at[idx_v.at[b]], bufs[b], sems[b])

    @pl.loop(0, NCHUNK // NBUF)
    def _grp(i):
        for b in range(NBUF):
            c = i * NBUF + b
            pltpu.make_async_copy(table_hbm.at[idx_v.at[c]],
                                  bufs[b], sems[b]).wait()
            _reduce_rows(bufs[b], c, out_v, gvec, bvec)

            @pl.when(c + NBUF < NCHUNK)
            def _prefetch():
                pltpu.async_copy(table_hbm.at[idx_v.at[c + NBUF]],
                                 bufs[b], sems[b])

    pltpu.sync_copy(out_v, out_hbm.at[pl.ds(wid * ROWS_PER_W * H, ROWS_PER_W * H)])


@functools.partial(jax.jit, static_argnames=())
def kernel(raw_features, dis_ids, table, gamma, beta):
    del dis_ids
    idx2d = raw_features.reshape(B * L // IDX_PER_CHUNK, IDX_PER_CHUNK)
    run = pl.kernel(
        _body,
        out_type=jax.ShapeDtypeStruct((B * H,), jnp.float32),
        mesh=plsc.VectorSubcoreMesh(core_axis_name="c", subcore_axis_name="s"),
        scratch_types=(
            [pltpu.VMEM((NCHUNK, IDX_PER_CHUNK), jnp.int32)]
            + [pltpu.VMEM((IDX_PER_CHUNK, H), jnp.float32)] * NBUF
            + [pltpu.VMEM((ROWS_PER_W * H,), jnp.float32),
               pltpu.VMEM((H,), jnp.float32),
               pltpu.VMEM((H,), jnp.float32)]
            + [pltpu.SemaphoreType.DMA] * NBUF
        ),
        compiler_params=pltpu.CompilerParams(
            needs_layout_passes=False, use_tc_tiling_on_sc=False),
    )
    out = run(table, idx2d, gamma, beta)
    return out.reshape(B, H)


# speedup vs baseline: 8.9524x; 1.2329x over previous
import functools

import jax
import jax.numpy as jnp
from jax import lax
from jax.experimental import pallas as pl
from jax.experimental.pallas import tpu as pltpu
from jax.experimental.pallas import tpu_sc as plsc

VOCAB = 100000
H = 64
B = 4096
L = 50
EPS = 1e-12

NC = 2
NS = 16
NW = NC * NS
ROWS_PER_W = B // NW
NBUF = 4


def _lane_total(v):
    lane15 = jnp.full((16,), 15, jnp.int32)
    return plsc.cumsum(v).at[lane15].get(mode="promise_in_bounds")


def _rsqrt(v):
    i = plsc.bitcast(v, jnp.int32)
    y = plsc.bitcast(jnp.int32(0x5F3759DF) - (i >> 1), jnp.float32)
    half = v * 0.5
    for _ in range(3):
        y = y * (1.5 - half * y * y)
    return y


def _reduce_row(buf, r, out_v, gvec, bvec):
    acc = [buf[0, pl.ds(k * 16, 16)] for k in range(4)]
    for l in range(1, L):
        for k in range(4):
            acc[k] = acc[k] + buf[l, pl.ds(k * 16, 16)]
    s = (acc[0] + acc[1]) + (acc[2] + acc[3])
    mean = _lane_total(s) * (1.0 / H)
    cent = [a - mean for a in acc]
    sq = ((cent[0] * cent[0] + cent[1] * cent[1])
          + (cent[2] * cent[2] + cent[3] * cent[3]))
    var = _lane_total(sq) * (1.0 / H)
    rstd = _rsqrt(var + EPS)
    for k in range(4):
        out_v[r, pl.ds(k * 16, 16)] = cent[k] * rstd * gvec[k] + bvec[k]


def _body(table_hbm, idx_hbm, gamma_hbm, beta_hbm, out_hbm,
          idx_v, b0, b1, b2, b3, out_v, gamma_v, beta_v,
          s0, s1, s2, s3):
    wid = lax.axis_index("s") * NC + lax.axis_index("c")
    bufs = (b0, b1, b2, b3)
    sems = (s0, s1, s2, s3)

    pltpu.sync_copy(idx_hbm.at[pl.ds(wid * ROWS_PER_W, ROWS_PER_W)], idx_v)
    pltpu.sync_copy(gamma_hbm, gamma_v)
    pltpu.sync_copy(beta_hbm, beta_v)
    gvec = [gamma_v[pl.ds(k * 16, 16)] for k in range(4)]
    bvec = [beta_v[pl.ds(k * 16, 16)] for k in range(4)]

    for b in range(NBUF):
        pltpu.async_copy(table_hbm.at[idx_v.at[b]], bufs[b], sems[b])

    @pl.loop(0, ROWS_PER_W // NBUF)
    def _grp(i):
        for b in range(NBUF):
            r = i * NBUF + b
            pltpu.make_async_copy(table_hbm.at[idx_v.at[r]],
                                  bufs[b], sems[b]).wait()
            _reduce_row(bufs[b], r, out_v, gvec, bvec)

            @pl.when(r + NBUF < ROWS_PER_W)
            def _prefetch():
                pltpu.async_copy(table_hbm.at[idx_v.at[r + NBUF]],
                                 bufs[b], sems[b])

    pltpu.sync_copy(out_v, out_hbm.at[pl.ds(wid * ROWS_PER_W, ROWS_PER_W)])


@functools.partial(jax.jit, static_argnames=())
def kernel(raw_features, dis_ids, table, gamma, beta):
    del dis_ids
    run = pl.kernel(
        _body,
        out_type=jax.ShapeDtypeStruct((B, H), jnp.float32),
        mesh=plsc.VectorSubcoreMesh(core_axis_name="c", subcore_axis_name="s"),
        scratch_types=(
            [pltpu.VMEM((ROWS_PER_W, L), jnp.int32)]
            + [pltpu.VMEM((L, H), jnp.float32)] * NBUF
            + [pltpu.VMEM((ROWS_PER_W, H), jnp.float32),
               pltpu.VMEM((H,), jnp.float32),
               pltpu.VMEM((H,), jnp.float32)]
            + [pltpu.SemaphoreType.DMA] * NBUF
        ),
        compiler_params=pltpu.CompilerParams(
            needs_layout_passes=False, use_tc_tiling_on_sc=False),
    )
    return run(table, raw_features, gamma, beta)
